# baseline (speedup 1.0000x reference)
import functools

import jax
import jax.numpy as jnp
from jax import lax
from jax.experimental import pallas as pl
from jax.experimental.pallas import tpu as pltpu
from jax.experimental.pallas import tpu_sc as plsc

N_NODES = 10000
NPAD = 10240
N_EDGES = 320000
D = 128
NC, NS = 2, 16
NW = NC * NS
CHUNK = 128
NCHUNKS = 2560
E_PAD = NCHUNKS * CHUNK
NITER = NCHUNKS // NW
ROWS_PER_TILE = NPAD // NS
NSTG = ROWS_PER_TILE // CHUNK
DEG_W = 16

_MESH = plsc.VectorSubcoreMesh(
    core_axis_name="c", subcore_axis_name="s", num_cores=NC, num_subcores=NS
)


def _seg_body(feats, src, dst, zfeat, out,
              s0b, s1b, s2b, s3b, dst_all, rows_a, rows_b, agg_sh,
              si0, si1, si2, si3, sga, sgb):
    c = lax.axis_index("c")
    s = lax.axis_index("s")
    w = s * NC + c
    srcb = [s0b, s1b, s2b, s3b]
    isem = [si0, si1, si2, si3]
    gsem = [sga, sgb]
    rows = [rows_a, rows_b]

    def src_sl(i):
        return src.at[pl.ds((w * NITER + i) * CHUNK, CHUNK)]

    r0 = s * ROWS_PER_TILE
    pltpu.sync_copy(zfeat, rows_a)
    for j in range(NSTG):
        pltpu.sync_copy(rows_a, agg_sh.at[pl.ds(r0 + j * CHUNK, CHUNK)])
    pltpu.sync_copy(dst.at[w], dst_all)
    plsc.subcore_barrier()

    for r in range(4):
        pltpu.async_copy(src_sl(r), srcb[r], isem[r])

    def step(i, r):
        pltpu.make_async_copy(src_sl(i), srcb[r], isem[r]).wait()
        pltpu.async_copy(feats.at[srcb[r]], rows[r % 2], gsem[r % 2]).wait()
        pltpu.sync_copy(rows[r % 2], agg_sh.at[dst_all.at[i]], add=True)

        @pl.when(i + 4 < NITER)
        def _():
            pltpu.async_copy(src_sl(i + 4), srcb[r], isem[r])

    def qbody(q, carry):
        for r in range(4):
            step(4 * q + r, r)
        return carry

    lax.fori_loop(0, NITER // 4, qbody, 0)
    plsc.subcore_barrier()

    for j in range(NSTG):
        sl = pl.ds(r0 + j * CHUNK, CHUNK)
        pltpu.sync_copy(agg_sh.at[sl], rows_a)
        pltpu.sync_copy(rows_a, out.at[c, sl])


def _deg_body(dst, zfeat, ones_h, degout, dst_all, ones_v, stg, deg_sh, sem):
    c = lax.axis_index("c")
    s = lax.axis_index("s")
    w = s * NC + c
    r0 = s * ROWS_PER_TILE
    pltpu.sync_copy(zfeat, stg)
    for j in range(NSTG):
        pltpu.sync_copy(stg, deg_sh.at[pl.ds(r0 + j * CHUNK, CHUNK)])
    pltpu.sync_copy(ones_h, ones_v)
    pltpu.sync_copy(dst.at[w], dst_all)
    plsc.subcore_barrier()

    def body(i, carry):
        pltpu.sync_copy(ones_v, deg_sh.at[dst_all.at[i]], add=True)
        return carry

    lax.fori_loop(0, NITER, body, 0)
    plsc.subcore_barrier()
    for j in range(NSTG):
        sl = pl.ds(r0 + j * CHUNK, CHUNK)
        pltpu.sync_copy(deg_sh.at[sl], stg)
        pltpu.sync_copy(stg, degout.at[c, sl])


_sc_segsum = functools.partial(
    pl.kernel,
    _seg_body,
    out_type=jax.ShapeDtypeStruct((NC, NPAD, D), jnp.float32),
    mesh=_MESH,
    scratch_types=[
        pltpu.VMEM((CHUNK,), jnp.int32),
        pltpu.VMEM((CHUNK,), jnp.int32),
        pltpu.VMEM((CHUNK,), jnp.int32),
        pltpu.VMEM((CHUNK,), jnp.int32),
        pltpu.VMEM((NITER, CHUNK), jnp.int32),
        pltpu.VMEM((CHUNK, D), jnp.float32),
        pltpu.VMEM((CHUNK, D), jnp.float32),
        pltpu.VMEM_SHARED((NPAD, D), jnp.float32),
        pltpu.SemaphoreType.DMA,
        pltpu.SemaphoreType.DMA,
        pltpu.SemaphoreType.DMA,
        pltpu.SemaphoreType.DMA,
        pltpu.SemaphoreType.DMA,
        pltpu.SemaphoreType.DMA,
    ],
)()

_sc_deg = functools.partial(
    pl.kernel,
    _deg_body,
    out_type=jax.ShapeDtypeStruct((NC, NPAD, D), jnp.float32),
    mesh=_MESH,
    scratch_types=[
        pltpu.VMEM((NITER, CHUNK), jnp.int32),
        pltpu.VMEM((CHUNK, D), jnp.float32),
        pltpu.VMEM((CHUNK, D), jnp.float32),
        pltpu.VMEM_SHARED((NPAD, D), jnp.float32),
        pltpu.SemaphoreType.DMA,
    ],
)()


def _dense_body(relu, aggp, degp, x, wl, b, wr, o):
    p = aggp[0] + aggp[1]
    deg = (jnp.sum(degp[0], axis=1) + jnp.sum(degp[1], axis=1)) * (1.0 / D)
    deg = jnp.maximum(deg, 1.0)
    mean = p / deg[:, None]
    acc = lax.dot_general(mean, wl[...], (((1,), (1,)), ((), ())),
                          preferred_element_type=jnp.float32)
    acc += lax.dot_general(x[...], wr[...], (((1,), (1,)), ((), ())),
                           preferred_element_type=jnp.float32)
    acc += b[...]
    o[...] = jnp.maximum(acc, 0.0) if relu else acc


def _dense(aggp, degp, x, wl, b, wr, relu):
    grid_n = 10
    r = NPAD // grid_n
    return pl.pallas_call(
        functools.partial(_dense_body, relu),
        out_shape=jax.ShapeDtypeStruct((NPAD, D), jnp.float32),
        grid=(grid_n,),
        in_specs=[
            pl.BlockSpec((NC, r, D), lambda i: (0, i, 0)),
            pl.BlockSpec((NC, r, D), lambda i: (0, i, 0)),
            pl.BlockSpec((r, D), lambda i: (i, 0)),
            pl.BlockSpec((D, D), lambda i: (0, 0)),
            pl.BlockSpec((1, D), lambda i: (0, 0)),
            pl.BlockSpec((D, D), lambda i: (0, 0)),
        ],
        out_specs=pl.BlockSpec((r, D), lambda i: (i, 0)),
    )(aggp, degp, x, wl, b, wr)


def kernel(x, edge_index, W_l1, b_l1, W_r1, W_l2, b_l2, W_r2):
    src = edge_index[0].astype(jnp.int32)
    dst = edge_index[1].astype(jnp.int32)
    npad_e = E_PAD - src.shape[0]
    src = jnp.concatenate([src, jnp.full((npad_e,), N_NODES, jnp.int32)])
    dst = jnp.concatenate([dst, jnp.full((npad_e,), NPAD - 1, jnp.int32)])
    dst = dst.reshape(NW, NITER, CHUNK)
    xp = jnp.pad(x, ((0, NPAD - N_NODES), (0, 0)))
    zfeat = jnp.zeros((CHUNK, D), jnp.float32)
    ones_h = jnp.ones((CHUNK, D), jnp.float32)

    degp = _sc_deg(dst, zfeat, ones_h)
    aggp1 = _sc_segsum(xp, src, dst, zfeat)
    h = _dense(aggp1, degp, xp, W_l1, b_l1.reshape(1, -1), W_r1, relu=True)
    aggp2 = _sc_segsum(h, src, dst, zfeat)
    out = _dense(aggp2, degp, h, W_l2, b_l2.reshape(1, -1), W_r2, relu=False)
    return out[:N_NODES]


# speedup vs baseline: 2.3243x; 2.3243x over previous
import functools

import jax
import jax.numpy as jnp
from jax import lax
from jax.experimental import pallas as pl
from jax.experimental.pallas import tpu as pltpu
from jax.experimental.pallas import tpu_sc as plsc

N_NODES = 10000
NPAD = 10240
N_EDGES = 320000
D = 128
NC, NS = 2, 16
NW = NC * NS
CHUNK = 128
NCHUNKS = 2560
E_PAD = NCHUNKS * CHUNK
NITER = NCHUNKS // NW
ROWS_PER_TILE = NPAD // NS
NSTG = ROWS_PER_TILE // CHUNK
DEG_W = 16

_MESH = plsc.VectorSubcoreMesh(
    core_axis_name="c", subcore_axis_name="s", num_cores=NC, num_subcores=NS
)


def _seg_body(feats, src, dst, zfeat, out,
              s0b, s1b, s2b, s3b, dst_all, rows_a, rows_b, agg_sh,
              si0, si1, si2, si3, sga, sgb):
    c = lax.axis_index("c")
    s = lax.axis_index("s")
    w = s * NC + c
    srcb = [s0b, s1b, s2b, s3b]
    isem = [si0, si1, si2, si3]
    gsem = [sga, sgb]
    rows = [rows_a, rows_b]

    def src_sl(i):
        return src.at[pl.ds((w * NITER + i) * CHUNK, CHUNK)]

    r0 = s * ROWS_PER_TILE
    pltpu.sync_copy(zfeat, rows_a)
    for j in range(NSTG):
        pltpu.sync_copy(rows_a, agg_sh.at[pl.ds(r0 + j * CHUNK, CHUNK)])
    pltpu.sync_copy(dst.at[w], dst_all)
    plsc.subcore_barrier()

    for r in range(4):
        pltpu.async_copy(src_sl(r), srcb[r], isem[r])

    def step(i, r):
        pltpu.make_async_copy(src_sl(i), srcb[r], isem[r]).wait()
        pltpu.async_copy(feats.at[srcb[r]], rows[r % 2], gsem[r % 2]).wait()
        pltpu.sync_copy(rows[r % 2], agg_sh.at[dst_all.at[i]], add=True)

        @pl.when(i + 4 < NITER)
        def _():
            pltpu.async_copy(src_sl(i + 4), srcb[r], isem[r])

    def qbody(q, carry):
        for r in range(4):
            step(4 * q + r, r)
        return carry

    lax.fori_loop(0, NITER // 4, qbody, 0)
    plsc.subcore_barrier()

    for j in range(NSTG):
        sl = pl.ds(r0 + j * CHUNK, CHUNK)
        pltpu.sync_copy(agg_sh.at[sl], rows_a)
        pltpu.sync_copy(rows_a, out.at[c, sl])


def _deg_body(dst, zfeat, ones_h, degout, dst_all, ones_v, stg, deg_sh, sem):
    c = lax.axis_index("c")
    s = lax.axis_index("s")
    w = s * NC + c
    r0 = s * ROWS_PER_TILE
    pltpu.sync_copy(zfeat, stg)
    for j in range(NSTG):
        pltpu.sync_copy(stg, deg_sh.at[pl.ds(r0 + j * CHUNK, CHUNK)])
    pltpu.sync_copy(ones_h, ones_v)
    pltpu.sync_copy(dst.at[w], dst_all)
    plsc.subcore_barrier()

    def body(i, carry):
        pltpu.sync_copy(ones_v, deg_sh.at[dst_all.at[i]], add=True)
        return carry

    lax.fori_loop(0, NITER, body, 0)
    plsc.subcore_barrier()
    for j in range(NSTG):
        sl = pl.ds(r0 + j * CHUNK, CHUNK)
        pltpu.sync_copy(deg_sh.at[sl], stg)
        pltpu.sync_copy(stg, degout.at[c, sl])


_sc_segsum = functools.partial(
    pl.kernel,
    _seg_body,
    out_type=jax.ShapeDtypeStruct((NC, NPAD, D), jnp.float32),
    mesh=_MESH,
    scratch_types=[
        pltpu.VMEM((CHUNK,), jnp.int32),
        pltpu.VMEM((CHUNK,), jnp.int32),
        pltpu.VMEM((CHUNK,), jnp.int32),
        pltpu.VMEM((CHUNK,), jnp.int32),
        pltpu.VMEM((NITER, CHUNK), jnp.int32),
        pltpu.VMEM((CHUNK, D), jnp.float32),
        pltpu.VMEM((CHUNK, D), jnp.float32),
        pltpu.VMEM_SHARED((NPAD, D), jnp.float32),
        pltpu.SemaphoreType.DMA,
        pltpu.SemaphoreType.DMA,
        pltpu.SemaphoreType.DMA,
        pltpu.SemaphoreType.DMA,
        pltpu.SemaphoreType.DMA,
        pltpu.SemaphoreType.DMA,
    ],
)()

_sc_deg = functools.partial(
    pl.kernel,
    _deg_body,
    out_type=jax.ShapeDtypeStruct((NC, NPAD, D), jnp.float32),
    mesh=_MESH,
    scratch_types=[
        pltpu.VMEM((NITER, CHUNK), jnp.int32),
        pltpu.VMEM((CHUNK, D), jnp.float32),
        pltpu.VMEM((CHUNK, D), jnp.float32),
        pltpu.VMEM_SHARED((NPAD, D), jnp.float32),
        pltpu.SemaphoreType.DMA,
    ],
)()


def _dense_body(relu, aggp, degp, x, wl, b, wr, o):
    p = aggp[0] + aggp[1]
    deg = (jnp.sum(degp[0], axis=1) + jnp.sum(degp[1], axis=1)) * (1.0 / D)
    deg = jnp.maximum(deg, 1.0)
    mean = p / deg[:, None]
    acc = lax.dot_general(mean, wl[...], (((1,), (1,)), ((), ())),
                          preferred_element_type=jnp.float32)
    acc += lax.dot_general(x[...], wr[...], (((1,), (1,)), ((), ())),
                           preferred_element_type=jnp.float32)
    acc += b[...]
    o[...] = jnp.maximum(acc, 0.0) if relu else acc


def _dense(aggp, degp, x, wl, b, wr, relu):
    grid_n = 10
    r = NPAD // grid_n
    return pl.pallas_call(
        functools.partial(_dense_body, relu),
        out_shape=jax.ShapeDtypeStruct((NPAD, D), jnp.float32),
        grid=(grid_n,),
        in_specs=[
            pl.BlockSpec((NC, r, D), lambda i: (0, i, 0)),
            pl.BlockSpec((NC, r, D), lambda i: (0, i, 0)),
            pl.BlockSpec((r, D), lambda i: (i, 0)),
            pl.BlockSpec((D, D), lambda i: (0, 0)),
            pl.BlockSpec((1, D), lambda i: (0, 0)),
            pl.BlockSpec((D, D), lambda i: (0, 0)),
        ],
        out_specs=pl.BlockSpec((r, D), lambda i: (i, 0)),
    )(aggp, degp, x, wl, b, wr)


def kernel(x, edge_index, W_l1, b_l1, W_r1, W_l2, b_l2, W_r2):
    src = edge_index[0].astype(jnp.int32)
    dst = edge_index[1].astype(jnp.int32)
    npad_e = E_PAD - src.shape[0]
    pad_rows = N_NODES + jnp.arange(npad_e, dtype=jnp.int32) % (NPAD - N_NODES)
    src = jnp.concatenate([src, pad_rows])
    dst = jnp.concatenate([dst, pad_rows])
    dst = dst.reshape(NW, NITER, CHUNK)
    xp = jnp.pad(x, ((0, NPAD - N_NODES), (0, 0)))
    zfeat = jnp.zeros((CHUNK, D), jnp.float32)
    ones_h = jnp.ones((CHUNK, D), jnp.float32)

    degp = _sc_deg(dst, zfeat, ones_h)
    aggp1 = _sc_segsum(xp, src, dst, zfeat)
    h = _dense(aggp1, degp, xp, W_l1, b_l1.reshape(1, -1), W_r1, relu=True)
    aggp2 = _sc_segsum(h, src, dst, zfeat)
    out = _dense(aggp2, degp, h, W_l2, b_l2.reshape(1, -1), W_r2, relu=False)
    return out[:N_NODES]


# speedup vs baseline: 3.2326x; 1.3908x over previous
import functools

import jax
import jax.numpy as jnp
from jax import lax
from jax.experimental import pallas as pl
from jax.experimental.pallas import tpu as pltpu
from jax.experimental.pallas import tpu_sc as plsc

N_NODES = 10000
NPAD = 10240
N_EDGES = 320000
D = 128
NC, NS = 2, 16
NW = NC * NS
CHUNK = 128
NCHUNKS = 2560
E_PAD = NCHUNKS * CHUNK
NITER = NCHUNKS // NW
ROWS_PER_TILE = NPAD // NS
NSTG = ROWS_PER_TILE // CHUNK
DEG_W = 16

_MESH = plsc.VectorSubcoreMesh(
    core_axis_name="c", subcore_axis_name="s", num_cores=NC, num_subcores=NS
)


def _seg_body(feats, src, dst, zfeat, out,
              s0b, s1b, s2b, s3b, dst_all, rows_a, rows_b, agg_sh,
              si0, si1, si2, si3, sga, sgb):
    c = lax.axis_index("c")
    s = lax.axis_index("s")
    w = s * NC + c
    srcb = [s0b, s1b, s2b, s3b]
    isem = [si0, si1, si2, si3]
    gsem = [sga, sgb]
    rows = [rows_a, rows_b]

    def src_sl(i):
        return src.at[pl.ds((w * NITER + i) * CHUNK, CHUNK)]

    r0 = s * ROWS_PER_TILE
    pltpu.sync_copy(zfeat, rows_a)
    for j in range(NSTG):
        pltpu.sync_copy(rows_a, agg_sh.at[pl.ds(r0 + j * CHUNK, CHUNK)])
    pltpu.sync_copy(dst.at[w], dst_all)
    plsc.subcore_barrier()

    for r in range(4):
        pltpu.async_copy(src_sl(r), srcb[r], isem[r])
    pltpu.make_async_copy(src_sl(0), s0b, si0).wait()
    pltpu.async_copy(feats.at[s0b], rows_a, sga)

    def step(i, r):
        rn = (r + 1) % 4

        @pl.when(i + 1 < NITER)
        def _():
            pltpu.make_async_copy(src_sl(i + 1), srcb[rn], isem[rn]).wait()
            pltpu.async_copy(feats.at[srcb[rn]], rows[(r + 1) % 2],
                             gsem[(r + 1) % 2])

        pltpu.make_async_copy(feats.at[srcb[r]], rows[r % 2], gsem[r % 2]).wait()
        pltpu.sync_copy(rows[r % 2], agg_sh.at[dst_all.at[i]], add=True)

        @pl.when(i + 4 < NITER)
        def _():
            pltpu.async_copy(src_sl(i + 4), srcb[r], isem[r])

    def qbody(q, carry):
        for r in range(4):
            step(4 * q + r, r)
        return carry

    lax.fori_loop(0, NITER // 4, qbody, 0)
    plsc.subcore_barrier()

    for j in range(NSTG):
        sl = pl.ds(r0 + j * CHUNK, CHUNK)
        pltpu.sync_copy(agg_sh.at[sl], rows_a)
        pltpu.sync_copy(rows_a, out.at[c, sl])


def _deg_body(dst, zfeat, ones_h, degout, dst_all, ones_v, stg, deg_sh, sem):
    c = lax.axis_index("c")
    s = lax.axis_index("s")
    w = s * NC + c
    r0 = s * ROWS_PER_TILE
    pltpu.sync_copy(zfeat, stg)
    for j in range(NSTG):
        pltpu.sync_copy(stg, deg_sh.at[pl.ds(r0 + j * CHUNK, CHUNK)])
    pltpu.sync_copy(ones_h, ones_v)
    pltpu.sync_copy(dst.at[w], dst_all)
    plsc.subcore_barrier()

    def body(i, carry):
        pltpu.sync_copy(ones_v, deg_sh.at[dst_all.at[i]], add=True)
        return carry

    lax.fori_loop(0, NITER, body, 0)
    plsc.subcore_barrier()
    for j in range(NSTG):
        sl = pl.ds(r0 + j * CHUNK, CHUNK)
        pltpu.sync_copy(deg_sh.at[sl], stg)
        pltpu.sync_copy(stg, degout.at[c, sl])


_sc_segsum = functools.partial(
    pl.kernel,
    _seg_body,
    out_type=jax.ShapeDtypeStruct((NC, NPAD, D), jnp.float32),
    mesh=_MESH,
    scratch_types=[
        pltpu.VMEM((CHUNK,), jnp.int32),
        pltpu.VMEM((CHUNK,), jnp.int32),
        pltpu.VMEM((CHUNK,), jnp.int32),
        pltpu.VMEM((CHUNK,), jnp.int32),
        pltpu.VMEM((NITER, CHUNK), jnp.int32),
        pltpu.VMEM((CHUNK, D), jnp.float32),
        pltpu.VMEM((CHUNK, D), jnp.float32),
        pltpu.VMEM_SHARED((NPAD, D), jnp.float32),
        pltpu.SemaphoreType.DMA,
        pltpu.SemaphoreType.DMA,
        pltpu.SemaphoreType.DMA,
        pltpu.SemaphoreType.DMA,
        pltpu.SemaphoreType.DMA,
        pltpu.SemaphoreType.DMA,
    ],
)()

_sc_deg = functools.partial(
    pl.kernel,
    _deg_body,
    out_type=jax.ShapeDtypeStruct((NC, NPAD, D), jnp.float32),
    mesh=_MESH,
    scratch_types=[
        pltpu.VMEM((NITER, CHUNK), jnp.int32),
        pltpu.VMEM((CHUNK, D), jnp.float32),
        pltpu.VMEM((CHUNK, D), jnp.float32),
        pltpu.VMEM_SHARED((NPAD, D), jnp.float32),
        pltpu.SemaphoreType.DMA,
    ],
)()


def _dense_body(relu, aggp, degp, x, wl, b, wr, o):
    p = aggp[0] + aggp[1]
    deg = (jnp.sum(degp[0], axis=1) + jnp.sum(degp[1], axis=1)) * (1.0 / D)
    deg = jnp.maximum(deg, 1.0)
    mean = p / deg[:, None]
    acc = lax.dot_general(mean, wl[...], (((1,), (1,)), ((), ())),
                          preferred_element_type=jnp.float32)
    acc += lax.dot_general(x[...], wr[...], (((1,), (1,)), ((), ())),
                           preferred_element_type=jnp.float32)
    acc += b[...]
    o[...] = jnp.maximum(acc, 0.0) if relu else acc


def _dense(aggp, degp, x, wl, b, wr, relu):
    grid_n = 10
    r = NPAD // grid_n
    return pl.pallas_call(
        functools.partial(_dense_body, relu),
        out_shape=jax.ShapeDtypeStruct((NPAD, D), jnp.float32),
        grid=(grid_n,),
        in_specs=[
            pl.BlockSpec((NC, r, D), lambda i: (0, i, 0)),
            pl.BlockSpec((NC, r, D), lambda i: (0, i, 0)),
            pl.BlockSpec((r, D), lambda i: (i, 0)),
            pl.BlockSpec((D, D), lambda i: (0, 0)),
            pl.BlockSpec((1, D), lambda i: (0, 0)),
            pl.BlockSpec((D, D), lambda i: (0, 0)),
        ],
        out_specs=pl.BlockSpec((r, D), lambda i: (i, 0)),
    )(aggp, degp, x, wl, b, wr)


def kernel(x, edge_index, W_l1, b_l1, W_r1, W_l2, b_l2, W_r2):
    src = edge_index[0].astype(jnp.int32)
    dst = edge_index[1].astype(jnp.int32)
    npad_e = E_PAD - src.shape[0]
    pad_rows = N_NODES + jnp.arange(npad_e, dtype=jnp.int32) % (NPAD - N_NODES)
    src = jnp.concatenate([src, pad_rows])
    dst = jnp.concatenate([dst, pad_rows])
    dst = dst.reshape(NW, NITER, CHUNK)
    xp = jnp.pad(x, ((0, NPAD - N_NODES), (0, 0)))
    zfeat = jnp.zeros((CHUNK, D), jnp.float32)
    ones_h = jnp.ones((CHUNK, D), jnp.float32)

    degp = _sc_deg(dst, zfeat, ones_h)
    aggp1 = _sc_segsum(xp, src, dst, zfeat)
    h = _dense(aggp1, degp, xp, W_l1, b_l1.reshape(1, -1), W_r1, relu=True)
    aggp2 = _sc_segsum(h, src, dst, zfeat)
    out = _dense(aggp2, degp, h, W_l2, b_l2.reshape(1, -1), W_r2, relu=False)
    return out[:N_NODES]


# speedup vs baseline: 3.3157x; 1.0257x over previous
import functools

import jax
import jax.numpy as jnp
from jax import lax
from jax.experimental import pallas as pl
from jax.experimental.pallas import tpu as pltpu
from jax.experimental.pallas import tpu_sc as plsc

N_NODES = 10000
NPAD = 10240
N_EDGES = 320000
D = 128
NC, NS = 2, 16
NW = NC * NS
CHUNK = 128
NCHUNKS = 2560
E_PAD = NCHUNKS * CHUNK
NITER = NCHUNKS // NW
ROWS_PER_TILE = NPAD // NS
NSTG = ROWS_PER_TILE // CHUNK
DEG_W = 16

_MESH = plsc.VectorSubcoreMesh(
    core_axis_name="c", subcore_axis_name="s", num_cores=NC, num_subcores=NS
)


def _seg_body(with_deg, feats, src, dst, zfeat, *rest):
    if with_deg:
        (ones_h, out, degout, s0b, s1b, s2b, s3b, dst_all,
         rows_a, rows_b, agg_sh, si0, si1, si2, si3, sga, sgb) = rest
    else:
        ones_h = degout = None
        (out, s0b, s1b, s2b, s3b, dst_all,
         rows_a, rows_b, agg_sh, si0, si1, si2, si3, sga, sgb) = rest
    c = lax.axis_index("c")
    s = lax.axis_index("s")
    w = s * NC + c
    srcb = [s0b, s1b, s2b, s3b]
    isem = [si0, si1, si2, si3]
    gsem = [sga, sgb]
    rows = [rows_a, rows_b]

    def src_sl(i):
        return src.at[pl.ds((w * NITER + i) * CHUNK, CHUNK)]

    r0 = s * ROWS_PER_TILE
    pltpu.sync_copy(zfeat, rows_a)
    for j in range(NSTG):
        pltpu.sync_copy(rows_a, agg_sh.at[pl.ds(r0 + j * CHUNK, CHUNK)])
    pltpu.sync_copy(dst.at[w], dst_all)
    plsc.subcore_barrier()

    if with_deg:
        pltpu.sync_copy(ones_h, rows_b)

        def dbody(i, carry):
            pltpu.sync_copy(rows_b, agg_sh.at[dst_all.at[i]], add=True)
            return carry

        lax.fori_loop(0, NITER, dbody, 0)
        plsc.subcore_barrier()
        for j in range(NSTG):
            sl = pl.ds(r0 + j * CHUNK, CHUNK)
            pltpu.sync_copy(agg_sh.at[sl], rows_a)
            pltpu.sync_copy(rows_a, degout.at[c, sl])
        pltpu.sync_copy(zfeat, rows_a)
        for j in range(NSTG):
            pltpu.sync_copy(rows_a, agg_sh.at[pl.ds(r0 + j * CHUNK, CHUNK)])
        plsc.subcore_barrier()

    for r in range(4):
        pltpu.async_copy(src_sl(r), srcb[r], isem[r])
    pltpu.make_async_copy(src_sl(0), s0b, si0).wait()
    pltpu.async_copy(feats.at[s0b], rows_a, sga)

    def step(i, r):
        rn = (r + 1) % 4

        @pl.when(i + 1 < NITER)
        def _():
            pltpu.make_async_copy(src_sl(i + 1), srcb[rn], isem[rn]).wait()
            pltpu.async_copy(feats.at[srcb[rn]], rows[(r + 1) % 2],
                             gsem[(r + 1) % 2])

        pltpu.make_async_copy(feats.at[srcb[r]], rows[r % 2], gsem[r % 2]).wait()
        pltpu.sync_copy(rows[r % 2], agg_sh.at[dst_all.at[i]], add=True)

        @pl.when(i + 4 < NITER)
        def _():
            pltpu.async_copy(src_sl(i + 4), srcb[r], isem[r])

    def qbody(q, carry):
        for r in range(4):
            step(4 * q + r, r)
        return carry

    lax.fori_loop(0, NITER // 4, qbody, 0)
    plsc.subcore_barrier()

    for j in range(NSTG):
        sl = pl.ds(r0 + j * CHUNK, CHUNK)
        pltpu.sync_copy(agg_sh.at[sl], rows_a)
        pltpu.sync_copy(rows_a, out.at[c, sl])


_SEG_SCRATCH = [
    pltpu.VMEM((CHUNK,), jnp.int32),
    pltpu.VMEM((CHUNK,), jnp.int32),
    pltpu.VMEM((CHUNK,), jnp.int32),
    pltpu.VMEM((CHUNK,), jnp.int32),
    pltpu.VMEM((NITER, CHUNK), jnp.int32),
    pltpu.VMEM((CHUNK, D), jnp.float32),
    pltpu.VMEM((CHUNK, D), jnp.float32),
    pltpu.VMEM_SHARED((NPAD, D), jnp.float32),
    pltpu.SemaphoreType.DMA,
    pltpu.SemaphoreType.DMA,
    pltpu.SemaphoreType.DMA,
    pltpu.SemaphoreType.DMA,
    pltpu.SemaphoreType.DMA,
    pltpu.SemaphoreType.DMA,
]

_sc_segsum_deg = functools.partial(
    pl.kernel,
    functools.partial(_seg_body, True),
    out_type=(
        jax.ShapeDtypeStruct((NC, NPAD, D), jnp.float32),
        jax.ShapeDtypeStruct((NC, NPAD, D), jnp.float32),
    ),
    mesh=_MESH,
    scratch_types=_SEG_SCRATCH,
)()

_sc_segsum = functools.partial(
    pl.kernel,
    functools.partial(_seg_body, False),
    out_type=jax.ShapeDtypeStruct((NC, NPAD, D), jnp.float32),
    mesh=_MESH,
    scratch_types=[
        pltpu.VMEM((CHUNK,), jnp.int32),
        pltpu.VMEM((CHUNK,), jnp.int32),
        pltpu.VMEM((CHUNK,), jnp.int32),
        pltpu.VMEM((CHUNK,), jnp.int32),
        pltpu.VMEM((NITER, CHUNK), jnp.int32),
        pltpu.VMEM((CHUNK, D), jnp.float32),
        pltpu.VMEM((CHUNK, D), jnp.float32),
        pltpu.VMEM_SHARED((NPAD, D), jnp.float32),
        pltpu.SemaphoreType.DMA,
        pltpu.SemaphoreType.DMA,
        pltpu.SemaphoreType.DMA,
        pltpu.SemaphoreType.DMA,
        pltpu.SemaphoreType.DMA,
        pltpu.SemaphoreType.DMA,
    ],
)()

def _dense_body(relu, aggp, degp, x, wl, b, wr, o):
    p = aggp[0] + aggp[1]
    deg = (jnp.sum(degp[0], axis=1) + jnp.sum(degp[1], axis=1)) * (1.0 / D)
    deg = jnp.maximum(deg, 1.0)
    mean = p / deg[:, None]
    acc = lax.dot_general(mean, wl[...], (((1,), (1,)), ((), ())),
                          preferred_element_type=jnp.float32)
    acc += lax.dot_general(x[...], wr[...], (((1,), (1,)), ((), ())),
                           preferred_element_type=jnp.float32)
    acc += b[...]
    o[...] = jnp.maximum(acc, 0.0) if relu else acc


def _dense(aggp, degp, x, wl, b, wr, relu, out_rows=NPAD):
    grid_n = 10
    r = out_rows // grid_n
    return pl.pallas_call(
        functools.partial(_dense_body, relu),
        out_shape=jax.ShapeDtypeStruct((out_rows, D), jnp.float32),
        grid=(grid_n,),
        in_specs=[
            pl.BlockSpec((NC, r, D), lambda i: (0, i, 0)),
            pl.BlockSpec((NC, r, D), lambda i: (0, i, 0)),
            pl.BlockSpec((r, D), lambda i: (i, 0)),
            pl.BlockSpec((D, D), lambda i: (0, 0)),
            pl.BlockSpec((1, D), lambda i: (0, 0)),
            pl.BlockSpec((D, D), lambda i: (0, 0)),
        ],
        out_specs=pl.BlockSpec((r, D), lambda i: (i, 0)),
    )(aggp, degp, x, wl, b, wr)


def kernel(x, edge_index, W_l1, b_l1, W_r1, W_l2, b_l2, W_r2):
    src = edge_index[0].astype(jnp.int32)
    dst = edge_index[1].astype(jnp.int32)
    npad_e = E_PAD - src.shape[0]
    pad_rows = N_NODES + jnp.arange(npad_e, dtype=jnp.int32) % (NPAD - N_NODES)
    src = jnp.concatenate([src, pad_rows])
    dst = jnp.concatenate([dst, pad_rows])
    dst = dst.reshape(NW, NITER, CHUNK)
    xp = jnp.pad(x, ((0, NPAD - N_NODES), (0, 0)))
    zfeat = jnp.zeros((CHUNK, D), jnp.float32)
    ones_h = jnp.ones((CHUNK, D), jnp.float32)

    aggp1, degp = _sc_segsum_deg(xp, src, dst, zfeat, ones_h)
    h = _dense(aggp1, degp, xp, W_l1, b_l1.reshape(1, -1), W_r1, relu=True)
    aggp2 = _sc_segsum(h, src, dst, zfeat)
    return _dense(aggp2, degp, h, W_l2, b_l2.reshape(1, -1), W_r2, relu=False,
                  out_rows=N_NODES)
